# Initial kernel scaffold; baseline (speedup 1.0000x reference)
import jax
import jax.numpy as jnp
from jax.experimental import pallas as pl


def kernel(x, edge_index, W1, b1, W2, b2):
    raise NotImplementedError("write your pallas kernel here")


# speedup vs baseline: 29.5828x; 29.5828x over previous
import functools

import jax
import jax.numpy as jnp
from jax import lax
from jax.experimental import pallas as pl
from jax.experimental.pallas import tpu as pltpu
from jax.experimental.pallas import tpu_sc as plsc

F32 = jnp.float32
CH = 128
BLK = 8

_MESH = plsc.VectorSubcoreMesh(core_axis_name="c", subcore_axis_name="s")


def _edge_pass1(srcb, dst2, xa, zeros16, n_pad, n_rows):
    rows_per_tile = n_rows // 32
    nblk = rows_per_tile // BLK
    zch = n_pad // 16

    def body(srcb_ref, dst_ref, xa_ref, z_ref, out_ref,
             acc, sidx, didx, rows, semg, sems):
        c = lax.axis_index("c")
        s = lax.axis_index("s")
        pltpu.sync_copy(z_ref.at[pl.ds(s * zch, zch)],
                        acc.at[pl.ds(s * zch, zch)])
        plsc.subcore_barrier()
        base = (c * 16 + s) * rows_per_tile

        def blk(b, carry):
            r0 = base + b * BLK
            pltpu.sync_copy(srcb_ref.at[0, pl.ds(r0, BLK)], sidx)
            pltpu.sync_copy(dst_ref.at[pl.ds(r0, BLK)], didx)
            g = [pltpu.async_copy(xa_ref.at[sidx.at[j]], rows.at[j], semg)
                 for j in range(BLK)]
            for d in g:
                d.wait()
            sc = [pltpu.async_copy(rows.at[j], acc.at[didx.at[j]], sems,
                                   add=True)
                  for j in range(BLK)]
            for d in sc:
                d.wait()
            return carry

        lax.fori_loop(0, nblk, blk, 0)
        plsc.subcore_barrier()
        pltpu.sync_copy(acc.at[pl.ds(s * zch, zch)],
                        out_ref.at[c, pl.ds(s * zch, zch)])

    return pl.kernel(
        body,
        out_type=jax.ShapeDtypeStruct((2, n_pad, 16), F32),
        mesh=_MESH,
        scratch_types=[
            pltpu.VMEM_SHARED((n_pad, 16), F32),
            pltpu.VMEM((BLK, CH), jnp.int32),
            pltpu.VMEM((BLK, CH), jnp.int32),
            pltpu.VMEM((BLK, CH, 16), F32),
            pltpu.SemaphoreType.DMA,
            pltpu.SemaphoreType.DMA,
        ],
        compiler_params=pltpu.CompilerParams(use_tc_tiling_on_sc=False),
    )(srcb, dst2, xa, zeros16)


def _edge_pass2(srcb, dst2, tbl, zeros16, n_pad, n_rows):
    rows_per_tile = n_rows // 16
    nblk = rows_per_tile // BLK
    zch = n_pad // 16

    def body(srcb_ref, dst_ref, tbl_ref, z_ref, out_ref,
             acc, sidx, didx, rows, semg, sems):
        c = lax.axis_index("c")
        s = lax.axis_index("s")
        pltpu.sync_copy(z_ref.at[pl.ds(s * zch, zch)],
                        acc.at[pl.ds(s * zch, zch)])
        plsc.subcore_barrier()
        base = s * rows_per_tile

        def blk(b, carry):
            r0 = base + b * BLK
            pltpu.sync_copy(srcb_ref.at[c, pl.ds(r0, BLK)], sidx)
            pltpu.sync_copy(dst_ref.at[pl.ds(r0, BLK)], didx)
            g = [pltpu.async_copy(tbl_ref.at[sidx.at[j]], rows.at[j], semg)
                 for j in range(BLK)]
            for d in g:
                d.wait()
            sc = [pltpu.async_copy(rows.at[j], acc.at[didx.at[j]], sems,
                                   add=True)
                  for j in range(BLK)]
            for d in sc:
                d.wait()
            return carry

        lax.fori_loop(0, nblk, blk, 0)
        plsc.subcore_barrier()
        pltpu.sync_copy(acc.at[pl.ds(s * zch, zch)],
                        out_ref.at[c, pl.ds(s * zch, zch)])

    return pl.kernel(
        body,
        out_type=jax.ShapeDtypeStruct((2, n_pad, 16), F32),
        mesh=_MESH,
        scratch_types=[
            pltpu.VMEM_SHARED((n_pad, 16), F32),
            pltpu.VMEM((BLK, CH), jnp.int32),
            pltpu.VMEM((BLK, CH), jnp.int32),
            pltpu.VMEM((BLK, CH, 16), F32),
            pltpu.SemaphoreType.DMA,
            pltpu.SemaphoreType.DMA,
        ],
        compiler_params=pltpu.CompilerParams(use_tc_tiling_on_sc=False),
    )(srcb, dst2, tbl, zeros16)


def _tc1(part1, w1, b1, n, n_pad, rb):
    grid = n // rb

    def body(p_ref, w_ref, b_ref, h_ref, d_ref):
        p = p_ref[0] + p_ref[1]
        den = jnp.maximum(p[:, 4:5], 1.0)
        mean = p[:, 0:4] / den
        w = w_ref[...]
        h = (b_ref[...]
             + mean[:, 0:1] * w[0:1, :]
             + mean[:, 1:2] * w[1:2, :]
             + mean[:, 2:3] * w[2:3, :]
             + mean[:, 3:4] * w[3:4, :])
        h = jnp.maximum(h, 0.0)
        h_ref[0] = h[:, 0:16]
        h_ref[1] = h[:, 16:32]
        d_ref[...] = den

    return pl.pallas_call(
        body,
        grid=(grid,),
        in_specs=[
            pl.BlockSpec((2, rb, 16), lambda i: (0, i, 0)),
            pl.BlockSpec((4, 32), lambda i: (0, 0)),
            pl.BlockSpec((1, 32), lambda i: (0, 0)),
        ],
        out_specs=[
            pl.BlockSpec((2, rb, 16), lambda i: (0, i, 0)),
            pl.BlockSpec((rb, 1), lambda i: (i, 0)),
        ],
        out_shape=[
            jax.ShapeDtypeStruct((2, n_pad, 16), F32),
            jax.ShapeDtypeStruct((n_pad, 1), F32),
        ],
    )(part1, w1, b1)


def _tc2(part2, den, w2, b2, n, rb):
    grid = n // rb

    def body(p_ref, d_ref, w_ref, b_ref, o_ref):
        m = jnp.concatenate([p_ref[0], p_ref[1]], axis=1)
        t = jnp.dot(m, w_ref[...], preferred_element_type=F32)
        o_ref[...] = jnp.maximum(t / d_ref[...] + b_ref[...], 0.0)

    return pl.pallas_call(
        body,
        grid=(grid,),
        in_specs=[
            pl.BlockSpec((2, rb, 16), lambda i: (0, i, 0)),
            pl.BlockSpec((rb, 1), lambda i: (i, 0)),
            pl.BlockSpec((32, 32), lambda i: (0, 0)),
            pl.BlockSpec((1, 32), lambda i: (0, 0)),
        ],
        out_specs=pl.BlockSpec((rb, 32), lambda i: (i, 0)),
        out_shape=jax.ShapeDtypeStruct((n, 32), F32),
    )(part2, den, w2, b2)


def kernel(x, edge_index, W1, b1, W2, b2):
    n = x.shape[0]
    e = edge_index.shape[1]
    hid = W2.shape[1]

    n_pad = ((n // 6400) + 1) * 6400
    unit = CH * BLK * 32
    e_pad = ((e + unit - 1) // unit) * unit
    n_rows = e_pad // CH
    rb = 2000

    src = edge_index[0]
    dst = edge_index[1]
    pad = e_pad - e
    ar = jnp.arange(pad, dtype=jnp.int32)
    src_p = jnp.concatenate([src, ar % n])
    dst_p = jnp.concatenate([dst, n + (ar % (n_pad - n))])
    srcb = jnp.stack([src_p, src_p + n_pad]).reshape(2, n_rows, CH)
    dst2 = dst_p.reshape(n_rows, CH)
    xa = (jnp.zeros((n_pad, 16), F32)
          .at[:n, :4].set(x)
          .at[:n, 4].set(1.0))
    z16 = jnp.zeros((n_pad, 16), F32)

    part1 = _edge_pass1(srcb, dst2, xa, z16, n_pad, n_rows)
    h1t, den = _tc1(part1, W1, b1.reshape(1, hid), n, n_pad, rb)
    tbl = h1t.reshape(2 * n_pad, 16)
    part2 = _edge_pass2(srcb, dst2, tbl, z16, n_pad, n_rows)
    return _tc2(part2, den, W2, b2.reshape(1, hid), n, rb)


# speedup vs baseline: 43.9289x; 1.4849x over previous
import jax
import jax.numpy as jnp
from jax import lax
from jax.experimental import pallas as pl
from jax.experimental.pallas import tpu as pltpu
from jax.experimental.pallas import tpu_sc as plsc

F32 = jnp.float32
CH = 128
BLK = 6

_MESH = plsc.VectorSubcoreMesh(core_axis_name="c", subcore_axis_name="s")


def _edge_pass(src2d, dst2d, tbl3, zeros, n_pad, n_rows, feat, split_edges):
    total_blocks = n_rows // BLK
    nblk = total_blocks // (32 if split_edges else 16)
    zch = n_pad // 16

    def body(src_ref, dst_ref, tbl_ref, z_ref, out_ref,
             acc, sidx, didx, rows2, semi, semg, sems):
        c = lax.axis_index("c")
        s = lax.axis_index("s")
        g = 0 if split_edges else c
        pltpu.sync_copy(z_ref.at[pl.ds(s * zch, zch)],
                        acc.at[pl.ds(s * zch, zch)])
        plsc.subcore_barrier()
        if split_edges:
            base = (c * 16 + s) * nblk
        else:
            base = s * nblk

        pltpu.async_copy(src_ref.at[base], sidx.at[0], semi)
        pltpu.async_copy(dst_ref.at[base], didx.at[0], semi)

        def blk(b, carry):
            slot2 = lax.rem(b, 2)
            slot3 = lax.rem(b, 3)

            @pl.when(b >= 2)
            def _():
                pltpu.make_async_copy(
                    rows2.at[0], acc.at[didx.at[0]], sems).wait()

            @pl.when(b + 1 < nblk)
            def _():
                r1 = base + b + 1
                nslot = lax.rem(b + 1, 3)
                pltpu.async_copy(src_ref.at[r1], sidx.at[nslot], semi)
                pltpu.async_copy(dst_ref.at[r1], didx.at[nslot], semi)

            pltpu.make_async_copy(src_ref.at[0], sidx.at[0], semi).wait()
            pltpu.make_async_copy(dst_ref.at[0], didx.at[0], semi).wait()

            pltpu.async_copy(tbl_ref.at[g].at[sidx.at[slot3]],
                             rows2.at[slot2], semg).wait()
            pltpu.async_copy(rows2.at[slot2], acc.at[didx.at[slot3]],
                             sems, add=True)
            return carry

        lax.fori_loop(0, nblk, blk, 0)
        pltpu.make_async_copy(rows2.at[0], acc.at[didx.at[0]], sems).wait()
        pltpu.make_async_copy(rows2.at[0], acc.at[didx.at[0]], sems).wait()
        plsc.subcore_barrier()
        pltpu.sync_copy(acc.at[pl.ds(s * zch, zch)],
                        out_ref.at[c, pl.ds(s * zch, zch)])

    return pl.kernel(
        body,
        out_type=jax.ShapeDtypeStruct((2, n_pad, feat), F32),
        mesh=_MESH,
        scratch_types=[
            pltpu.VMEM_SHARED((n_pad, feat), F32),
            pltpu.VMEM((3, BLK * CH), jnp.int32),
            pltpu.VMEM((3, BLK * CH), jnp.int32),
            pltpu.VMEM((2, BLK * CH, feat), F32),
            pltpu.SemaphoreType.DMA,
            pltpu.SemaphoreType.DMA,
            pltpu.SemaphoreType.DMA,
        ],
        compiler_params=pltpu.CompilerParams(use_tc_tiling_on_sc=False),
    )(src2d, dst2d, tbl3, zeros)


def _tc1(part1, w1, b1, n, n_pad, rb):
    grid = n // rb

    def body(p_ref, w_ref, b_ref, h_ref, d_ref):
        p = p_ref[0] + p_ref[1]
        den = jnp.maximum(p[:, 4:5], 1.0)
        mean = p[:, 0:4] / den
        w = w_ref[...]
        h = (b_ref[...]
             + mean[:, 0:1] * w[0:1, :]
             + mean[:, 1:2] * w[1:2, :]
             + mean[:, 2:3] * w[2:3, :]
             + mean[:, 3:4] * w[3:4, :])
        h = jnp.maximum(h, 0.0)
        h_ref[0] = h[:, 0:16]
        h_ref[1] = h[:, 16:32]
        d_ref[...] = den

    return pl.pallas_call(
        body,
        grid=(grid,),
        in_specs=[
            pl.BlockSpec((2, rb, 8), lambda i: (0, i, 0)),
            pl.BlockSpec((4, 32), lambda i: (0, 0)),
            pl.BlockSpec((1, 32), lambda i: (0, 0)),
        ],
        out_specs=[
            pl.BlockSpec((2, rb, 16), lambda i: (0, i, 0)),
            pl.BlockSpec((rb, 1), lambda i: (i, 0)),
        ],
        out_shape=[
            jax.ShapeDtypeStruct((2, n_pad, 16), F32),
            jax.ShapeDtypeStruct((n_pad, 1), F32),
        ],
    )(part1, w1, b1)


def _tc2(part2, den, w2, b2, n, rb):
    grid = n // rb

    def body(p_ref, d_ref, w_ref, b_ref, o_ref):
        m = jnp.concatenate([p_ref[0], p_ref[1]], axis=1)
        t = jnp.dot(m, w_ref[...], preferred_element_type=F32)
        o_ref[...] = jnp.maximum(t / d_ref[...] + b_ref[...], 0.0)

    return pl.pallas_call(
        body,
        grid=(grid,),
        in_specs=[
            pl.BlockSpec((2, rb, 16), lambda i: (0, i, 0)),
            pl.BlockSpec((rb, 1), lambda i: (i, 0)),
            pl.BlockSpec((32, 32), lambda i: (0, 0)),
            pl.BlockSpec((1, 32), lambda i: (0, 0)),
        ],
        out_specs=pl.BlockSpec((rb, 32), lambda i: (i, 0)),
        out_shape=jax.ShapeDtypeStruct((n, 32), F32),
    )(part2, den, w2, b2)


def kernel(x, edge_index, W1, b1, W2, b2):
    n = x.shape[0]
    e = edge_index.shape[1]
    hid = W2.shape[1]

    n_pad = ((n // 1600) + 1) * 1600
    unit = CH * BLK * 32
    e_pad = ((e + unit - 1) // unit) * unit
    n_rows = e_pad // CH
    rb = 2000

    src = edge_index[0]
    dst = edge_index[1]
    pad = e_pad - e
    ar = jnp.arange(pad, dtype=jnp.int32)
    src2d = jnp.concatenate([src, ar % n]).reshape(-1, CH * BLK)
    dst2d = (jnp.concatenate([dst, n + (ar % (n_pad - n))])
             .reshape(-1, CH * BLK))
    xa = (jnp.zeros((1, n_pad, 8), F32)
          .at[0, :n, :4].set(x)
          .at[0, :n, 4].set(1.0))
    z8 = jnp.zeros((n_pad, 8), F32)
    z16 = jnp.zeros((n_pad, 16), F32)

    part1 = _edge_pass(src2d, dst2d, xa, z8, n_pad, n_rows, 8, True)
    h1t, den = _tc1(part1, W1, b1.reshape(1, hid), n, n_pad, rb)
    part2 = _edge_pass(src2d, dst2d, h1t, z16, n_pad, n_rows, 16, False)
    return _tc2(part2, den, W2, b2.reshape(1, hid), n, rb)


# speedup vs baseline: 50.0762x; 1.1399x over previous
import jax
import jax.numpy as jnp
from jax import lax
from jax.experimental import pallas as pl
from jax.experimental.pallas import tpu as pltpu
from jax.experimental.pallas import tpu_sc as plsc

F32 = jnp.float32
CH = 128
BLK = 4

_MESH = plsc.VectorSubcoreMesh(core_axis_name="c", subcore_axis_name="s")


def _edge_pass(src2d, dst2d, tbl3, zeros, n_pad, n_rows, feat, split_edges, bk):
    total_blocks = n_rows // bk
    nblk = total_blocks // (32 if split_edges else 16)
    zch = n_pad // 16

    def body(src_ref, dst_ref, tbl_ref, z_ref, out_ref,
             acc, sidx, didx, rows3, semi, semg, sems):
        c = lax.axis_index("c")
        s = lax.axis_index("s")
        g = 0 if split_edges else c
        pltpu.sync_copy(z_ref.at[pl.ds(s * zch, zch)],
                        acc.at[pl.ds(s * zch, zch)])
        plsc.subcore_barrier()
        if split_edges:
            base = (c * 16 + s) * nblk
        else:
            base = s * nblk

        pltpu.async_copy(src_ref.at[base], sidx.at[0], semi)
        pltpu.async_copy(dst_ref.at[base], didx.at[0], semi)
        if nblk > 1:
            pltpu.async_copy(src_ref.at[base + 1], sidx.at[1], semi)
            pltpu.async_copy(dst_ref.at[base + 1], didx.at[1], semi)
        pltpu.make_async_copy(src_ref.at[0], sidx.at[0], semi).wait()
        pltpu.make_async_copy(dst_ref.at[0], didx.at[0], semi).wait()
        pltpu.async_copy(tbl_ref.at[g].at[sidx.at[0]], rows3.at[0], semg)

        def blk(b, carry):
            s3 = lax.rem(b, 3)
            s4 = lax.rem(b, 4)

            @pl.when(b >= 2)
            def _():
                pltpu.make_async_copy(
                    rows3.at[0], acc.at[didx.at[0]], sems).wait()

            @pl.when(b + 2 < nblk)
            def _():
                r2 = base + b + 2
                pltpu.async_copy(src_ref.at[r2],
                                 sidx.at[lax.rem(b + 2, 3)], semi)
                pltpu.async_copy(dst_ref.at[r2],
                                 didx.at[lax.rem(b + 2, 4)], semi)

            @pl.when(b + 1 < nblk)
            def _():
                pltpu.make_async_copy(src_ref.at[0], sidx.at[0],
                                      semi).wait()
                pltpu.make_async_copy(dst_ref.at[0], didx.at[0],
                                      semi).wait()
                pltpu.async_copy(tbl_ref.at[g].at[sidx.at[lax.rem(b + 1, 3)]],
                                 rows3.at[lax.rem(b + 1, 3)], semg)

            pltpu.make_async_copy(tbl_ref.at[g].at[sidx.at[0]],
                                  rows3.at[0], semg).wait()
            pltpu.async_copy(rows3.at[s3], acc.at[didx.at[s4]],
                             sems, add=True)
            return carry

        lax.fori_loop(0, nblk, blk, 0)
        pltpu.make_async_copy(rows3.at[0], acc.at[didx.at[0]], sems).wait()
        pltpu.make_async_copy(rows3.at[0], acc.at[didx.at[0]], sems).wait()
        plsc.subcore_barrier()
        if split_edges:
            pltpu.sync_copy(acc.at[pl.ds(s * zch, zch)],
                            out_ref.at[c, pl.ds(s * zch, zch)])
        else:
            pltpu.sync_copy(acc.at[pl.ds(s * zch, zch)],
                            out_ref.at[pl.ds(s * zch, zch),
                                       pl.ds(c * feat, feat)])

    out_shape = ((2, n_pad, feat) if split_edges else (n_pad, 2 * feat))
    return pl.kernel(
        body,
        out_type=jax.ShapeDtypeStruct(out_shape, F32),
        mesh=_MESH,
        scratch_types=[
            pltpu.VMEM_SHARED((n_pad, feat), F32),
            pltpu.VMEM((3, bk * CH), jnp.int32),
            pltpu.VMEM((4, bk * CH), jnp.int32),
            pltpu.VMEM((3, bk * CH, feat), F32),
            pltpu.SemaphoreType.DMA,
            pltpu.SemaphoreType.DMA,
            pltpu.SemaphoreType.DMA,
        ],
        compiler_params=pltpu.CompilerParams(use_tc_tiling_on_sc=False),
    )(src2d, dst2d, tbl3, zeros)


def _sc_transform(part1, w1f, b1f, n, nt):
    npt = nt // 32
    cchunk = npt // 4
    ngrp = cchunk // 16

    def body(part_ref, w_ref, b_ref, h1t_ref, inv_ref,
             p0b, p1b, hb, ib, wb, bb):
        c = lax.axis_index("c")
        s = lax.axis_index("s")
        node0 = (c * 16 + s) * npt
        pltpu.sync_copy(w_ref, wb)
        pltpu.sync_copy(b_ref, bb)
        lane = jnp.arange(16, dtype=jnp.int32)
        for ch in range(4):
            base = node0 + ch * cchunk
            pltpu.sync_copy(part_ref.at[0, pl.ds(base, cchunk)], p0b)
            pltpu.sync_copy(part_ref.at[1, pl.ds(base, cchunk)], p1b)

            def grp(gi, carry):
                off = gi * 16
                rows = off + lane
                col = [plsc.load_gather(p0b, [rows, lane * 0 + k])
                       + plsc.load_gather(p1b, [rows, lane * 0 + k])
                       for k in range(5)]
                den = jnp.maximum(col[4], 1.0)
                inv = 1.0 / den
                mk = [col[k] * inv for k in range(4)]
                vmask = ((base + off) < n).astype(F32)
                for j in range(32):
                    h = (bb[j]
                         + mk[0] * wb[0, j]
                         + mk[1] * wb[1, j]
                         + mk[2] * wb[2, j]
                         + mk[3] * wb[3, j])
                    h = jnp.maximum(h, 0.0) * vmask
                    plsc.store_scatter(hb, [rows, lane * 0 + j], h)
                    plsc.store_scatter(ib, [rows, lane * 0 + j], inv)
                return carry

            lax.fori_loop(0, ngrp, grp, 0)
            pltpu.sync_copy(hb.at[:, pl.ds(0, 16)],
                            h1t_ref.at[0, pl.ds(base, cchunk)])
            pltpu.sync_copy(hb.at[:, pl.ds(16, 16)],
                            h1t_ref.at[1, pl.ds(base, cchunk)])
            pltpu.sync_copy(ib, inv_ref.at[pl.ds(base, cchunk)])

    return pl.kernel(
        body,
        out_type=[
            jax.ShapeDtypeStruct((2, nt, 16), F32),
            jax.ShapeDtypeStruct((nt, 32), F32),
        ],
        mesh=_MESH,
        scratch_types=[
            pltpu.VMEM((cchunk, 8), F32),
            pltpu.VMEM((cchunk, 8), F32),
            pltpu.VMEM((cchunk, 32), F32),
            pltpu.VMEM((cchunk, 32), F32),
            pltpu.VMEM((4, 32, 16), F32),
            pltpu.VMEM((32, 16), F32),
        ],
        compiler_params=pltpu.CompilerParams(
            use_tc_tiling_on_sc=False, needs_layout_passes=False),
    )(part1, w1f, b1f)


def _tc2(m_p, inv_p, w2bd, b2q, n):
    rows = n * 32 // 128
    rbp = 1000
    grid = rows // rbp

    def body(m_ref, i_ref, w_ref, b_ref, o_ref):
        t = jnp.dot(m_ref[...], w_ref[...], preferred_element_type=F32)
        o_ref[...] = jnp.maximum(t * i_ref[...] + b_ref[...], 0.0)

    return pl.pallas_call(
        body,
        grid=(grid,),
        in_specs=[
            pl.BlockSpec((rbp, 128), lambda i: (i, 0)),
            pl.BlockSpec((rbp, 128), lambda i: (i, 0)),
            pl.BlockSpec((128, 128), lambda i: (0, 0)),
            pl.BlockSpec((1, 128), lambda i: (0, 0)),
        ],
        out_specs=pl.BlockSpec((rbp, 128), lambda i: (i, 0)),
        out_shape=jax.ShapeDtypeStruct((rows, 128), F32),
    )(m_p, inv_p, w2bd, b2q)


def kernel(x, edge_index, W1, b1, W2, b2):
    n = x.shape[0]
    e = edge_index.shape[1]

    nt = ((n // 3136) + 1) * 3136
    unit = CH * 8 * 32
    e_pad = ((e + unit - 1) // unit) * unit
    n_rows = e_pad // CH

    src = edge_index[0]
    dst = edge_index[1]
    pad = e_pad - e
    padi = n + (jnp.arange(pad, dtype=jnp.int32) % (nt - n))
    src_p = jnp.concatenate([src, padi])
    dst_p = jnp.concatenate([dst, padi])
    src8 = src_p.reshape(-1, CH * 8)
    dst8 = dst_p.reshape(-1, CH * 8)
    src4 = src_p.reshape(-1, CH * 4)
    dst4 = dst_p.reshape(-1, CH * 4)
    xa = (jnp.zeros((1, nt, 8), F32)
          .at[0, :n, :4].set(x)
          .at[0, :n, 4].set(1.0))
    z8 = jnp.zeros((nt, 8), F32)
    z16 = jnp.zeros((nt, 16), F32)

    part1 = _edge_pass(src8, dst8, xa, z8, nt, n_rows, 8, True, 8)
    w1b = jnp.broadcast_to(W1[:, :, None], (4, 32, 16))
    b1b = jnp.broadcast_to(b1[:, None], (32, 16))
    h1t, inv = _sc_transform(part1, w1b, b1b, n, nt)
    part2 = _edge_pass(src4, dst4, h1t, z16, nt, n_rows, 16, False, 4)
    m_p = part2.reshape(-1, 128)
    inv_p = inv.reshape(-1, 128)
    w2bd = jnp.kron(jnp.eye(4, dtype=F32), W2)
    b2q = jnp.tile(b2, 4).reshape(1, 128)
    out = _tc2(m_p, inv_p, w2bd, b2q, n)
    return out.reshape(n, 32)


# speedup vs baseline: 52.0536x; 1.0395x over previous
import jax
import jax.numpy as jnp
from jax import lax
from jax.experimental import pallas as pl
from jax.experimental.pallas import tpu as pltpu
from jax.experimental.pallas import tpu_sc as plsc

F32 = jnp.float32
CH = 128
BLK = 4

_MESH = plsc.VectorSubcoreMesh(core_axis_name="c", subcore_axis_name="s")


def _edge_pass(src2d, dst2d, tbl3, zeros, n_pad, n_rows, feat, split_edges, bk):
    total_blocks = n_rows // bk
    nblk = total_blocks // (32 if split_edges else 16)
    zch = n_pad // 16

    def body(src_ref, dst_ref, tbl_ref, z_ref, out_ref,
             acc, sidx, didx, rows3, semi, semg, sems):
        c = lax.axis_index("c")
        s = lax.axis_index("s")
        g = 0 if split_edges else c
        pltpu.sync_copy(z_ref.at[pl.ds(s * zch, zch)],
                        acc.at[pl.ds(s * zch, zch)])
        plsc.subcore_barrier()
        if split_edges:
            base = (c * 16 + s) * nblk
        else:
            base = s * nblk

        pltpu.async_copy(src_ref.at[base], sidx.at[0], semi)
        pltpu.async_copy(dst_ref.at[base], didx.at[0], semi)
        if nblk > 1:
            pltpu.async_copy(src_ref.at[base + 1], sidx.at[1], semi)
            pltpu.async_copy(dst_ref.at[base + 1], didx.at[1], semi)
        pltpu.make_async_copy(src_ref.at[0], sidx.at[0], semi).wait()
        pltpu.make_async_copy(dst_ref.at[0], didx.at[0], semi).wait()
        pltpu.async_copy(tbl_ref.at[g].at[sidx.at[0]], rows3.at[0], semg)

        def blk(b, carry):
            s3 = lax.rem(b, 3)
            s4 = lax.rem(b, 4)

            @pl.when(b >= 2)
            def _():
                pltpu.make_async_copy(
                    rows3.at[0], acc.at[didx.at[0]], sems).wait()

            @pl.when(b + 2 < nblk)
            def _():
                r2 = base + b + 2
                pltpu.async_copy(src_ref.at[r2],
                                 sidx.at[lax.rem(b + 2, 3)], semi)
                pltpu.async_copy(dst_ref.at[r2],
                                 didx.at[lax.rem(b + 2, 4)], semi)

            @pl.when(b + 1 < nblk)
            def _():
                pltpu.make_async_copy(src_ref.at[0], sidx.at[0],
                                      semi).wait()
                pltpu.make_async_copy(dst_ref.at[0], didx.at[0],
                                      semi).wait()
                pltpu.async_copy(tbl_ref.at[g].at[sidx.at[lax.rem(b + 1, 3)]],
                                 rows3.at[lax.rem(b + 1, 3)], semg)

            pltpu.make_async_copy(tbl_ref.at[g].at[sidx.at[0]],
                                  rows3.at[0], semg).wait()
            pltpu.async_copy(rows3.at[s3], acc.at[didx.at[s4]],
                             sems, add=True)
            return carry

        lax.fori_loop(0, nblk, blk, 0)
        pltpu.make_async_copy(rows3.at[0], acc.at[didx.at[0]], sems).wait()
        pltpu.make_async_copy(rows3.at[0], acc.at[didx.at[0]], sems).wait()
        plsc.subcore_barrier()
        if split_edges:
            pltpu.sync_copy(acc.at[pl.ds(s * zch, zch)],
                            out_ref.at[c, pl.ds(s * zch, zch)])
        else:
            pltpu.sync_copy(acc.at[pl.ds(s * zch, zch)],
                            out_ref.at[pl.ds(s * zch, zch),
                                       pl.ds(c * feat, feat)])

    out_shape = ((2, n_pad, feat) if split_edges else (n_pad, 2 * feat))
    return pl.kernel(
        body,
        out_type=jax.ShapeDtypeStruct(out_shape, F32),
        mesh=_MESH,
        scratch_types=[
            pltpu.VMEM_SHARED((n_pad, feat), F32),
            pltpu.VMEM((3, bk * CH), jnp.int32),
            pltpu.VMEM((4, bk * CH), jnp.int32),
            pltpu.VMEM((3, bk * CH, feat), F32),
            pltpu.SemaphoreType.DMA,
            pltpu.SemaphoreType.DMA,
            pltpu.SemaphoreType.DMA,
        ],
        compiler_params=pltpu.CompilerParams(use_tc_tiling_on_sc=False),
    )(src2d, dst2d, tbl3, zeros)


def _sc_transform(part1, w1f, b1f, n, nt):
    npt = nt // 32
    cchunk = npt // 4
    ngrp = cchunk // 16

    def body(part_ref, w_ref, b_ref, h1t_ref, inv_ref,
             p0b, p1b, hb, ib, wb, bb):
        c = lax.axis_index("c")
        s = lax.axis_index("s")
        node0 = (c * 16 + s) * npt
        pltpu.sync_copy(w_ref, wb)
        pltpu.sync_copy(b_ref, bb)
        lane = jnp.arange(16, dtype=jnp.int32)
        for ch in range(4):
            base = node0 + ch * cchunk
            pltpu.sync_copy(part_ref.at[0, pl.ds(base, cchunk)], p0b)
            pltpu.sync_copy(part_ref.at[1, pl.ds(base, cchunk)], p1b)

            def grp(gi, carry):
                off = gi * 16
                rows = off + lane
                col = [plsc.load_gather(p0b, [rows, lane * 0 + k])
                       + plsc.load_gather(p1b, [rows, lane * 0 + k])
                       for k in range(5)]
                den = jnp.maximum(col[4], 1.0)
                inv = 1.0 / den
                mk = [col[k] * inv for k in range(4)]
                vmask = ((base + off) < n).astype(F32)
                for j in range(32):
                    h = (bb[j]
                         + mk[0] * wb[0, j]
                         + mk[1] * wb[1, j]
                         + mk[2] * wb[2, j]
                         + mk[3] * wb[3, j])
                    h = jnp.maximum(h, 0.0) * vmask
                    plsc.store_scatter(hb, [rows, lane * 0 + j], h)
                    plsc.store_scatter(ib, [rows, lane * 0 + j], inv)
                return carry

            lax.fori_loop(0, ngrp, grp, 0)
            pltpu.sync_copy(hb.at[:, pl.ds(0, 16)],
                            h1t_ref.at[0, pl.ds(base, cchunk)])
            pltpu.sync_copy(hb.at[:, pl.ds(16, 16)],
                            h1t_ref.at[1, pl.ds(base, cchunk)])
            pltpu.sync_copy(ib, inv_ref.at[pl.ds(base, cchunk)])

    return pl.kernel(
        body,
        out_type=[
            jax.ShapeDtypeStruct((2, nt, 16), F32),
            jax.ShapeDtypeStruct((nt, 32), F32),
        ],
        mesh=_MESH,
        scratch_types=[
            pltpu.VMEM((cchunk, 8), F32),
            pltpu.VMEM((cchunk, 8), F32),
            pltpu.VMEM((cchunk, 32), F32),
            pltpu.VMEM((cchunk, 32), F32),
            pltpu.VMEM((4, 32, 16), F32),
            pltpu.VMEM((32, 16), F32),
        ],
        compiler_params=pltpu.CompilerParams(
            use_tc_tiling_on_sc=False, needs_layout_passes=False),
    )(part1, w1f, b1f)


def _tc2(m_p, inv_p, w2bd, b2q, n):
    rows = n * 32 // 128
    rbp = 1000
    grid = rows // rbp

    def body(m_ref, i_ref, w_ref, b_ref, o_ref):
        t = jnp.dot(m_ref[...], w_ref[...], preferred_element_type=F32)
        o_ref[...] = jnp.maximum(t * i_ref[...] + b_ref[...], 0.0)

    return pl.pallas_call(
        body,
        grid=(grid,),
        in_specs=[
            pl.BlockSpec((rbp, 128), lambda i: (i, 0)),
            pl.BlockSpec((rbp, 128), lambda i: (i, 0)),
            pl.BlockSpec((128, 128), lambda i: (0, 0)),
            pl.BlockSpec((1, 128), lambda i: (0, 0)),
        ],
        out_specs=pl.BlockSpec((rbp, 128), lambda i: (i, 0)),
        out_shape=jax.ShapeDtypeStruct((rows, 128), F32),
    )(m_p, inv_p, w2bd, b2q)


def kernel(x, edge_index, W1, b1, W2, b2):
    n = x.shape[0]
    e = edge_index.shape[1]

    nt = ((n // 3136) + 1) * 3136
    unit = CH * 16 * 32
    e_pad = ((e + unit - 1) // unit) * unit
    n_rows = e_pad // CH

    src = edge_index[0]
    dst = edge_index[1]
    pad = e_pad - e
    padi = n + (jnp.arange(pad, dtype=jnp.int32) % (nt - n))
    src_p = jnp.concatenate([src, padi])
    dst_p = jnp.concatenate([dst, padi])
    src8 = src_p.reshape(-1, CH * 16)
    dst8 = dst_p.reshape(-1, CH * 16)
    src4 = src_p.reshape(-1, CH * 4)
    dst4 = dst_p.reshape(-1, CH * 4)
    xa = (jnp.zeros((1, nt, 8), F32)
          .at[0, :n, :4].set(x)
          .at[0, :n, 4].set(1.0))
    z8 = jnp.zeros((nt, 8), F32)
    z16 = jnp.zeros((nt, 16), F32)

    part1 = _edge_pass(src8, dst8, xa, z8, nt, n_rows, 8, True, 16)
    w1b = jnp.broadcast_to(W1[:, :, None], (4, 32, 16))
    b1b = jnp.broadcast_to(b1[:, None], (32, 16))
    h1t, inv = _sc_transform(part1, w1b, b1b, n, nt)
    part2 = _edge_pass(src4, dst4, h1t, z16, nt, n_rows, 16, False, 4)
    m_p = part2.reshape(-1, 128)
    inv_p = inv.reshape(-1, 128)
    w2bd = jnp.kron(jnp.eye(4, dtype=F32), W2)
    b2q = jnp.tile(b2, 4).reshape(1, 128)
    out = _tc2(m_p, inv_p, w2bd, b2q, n)
    return out.reshape(n, 32)
